# Initial kernel scaffold; baseline (speedup 1.0000x reference)
import jax
import jax.numpy as jnp
from jax.experimental import pallas as pl


def kernel(input_ids, segment_ids, token_table, pos_table, seg_table, gamma, beta):
    raise NotImplementedError("write your pallas kernel here")


# speedup vs baseline: 1.7084x; 1.7084x over previous
import functools

import jax
import jax.numpy as jnp
from jax import lax
from jax.experimental import pallas as pl
from jax.experimental.pallas import tpu as pltpu
from jax.experimental.pallas import tpu_sc as plsc


@functools.lru_cache(maxsize=None)
def _sc_gather(n_tokens: int, vocab: int, d: int):
    info = plsc.get_sparse_core_info()
    nw = info.num_cores * info.num_subcores
    per_w = n_tokens // nw
    ch = 64
    n_ch = per_w // ch
    mesh = plsc.VectorSubcoreMesh(core_axis_name="c", subcore_axis_name="s")

    @functools.partial(
        pl.kernel,
        mesh=mesh,
        out_type=jax.ShapeDtypeStruct((n_tokens, d), jnp.float32),
        scratch_types=[
            pltpu.VMEM((per_w,), jnp.int32),
            pltpu.VMEM((2, ch, d), jnp.float32),
            pltpu.SemaphoreType.DMA,
            pltpu.SemaphoreType.DMA,
        ],
    )
    def gather_kernel(ids_hbm, table_hbm, out_hbm, idx_v, rows_v, gsem, osem):
        wid = lax.axis_index("s") * info.num_cores + lax.axis_index("c")
        base = wid * per_w
        pltpu.sync_copy(ids_hbm.at[pl.ds(base, per_w)], idx_v)

        pltpu.async_copy(table_hbm.at[idx_v.at[pl.ds(0, ch)]], rows_v.at[0], gsem)

        def body(k, _):
            slot = lax.rem(k, 2)
            nxt = lax.rem(k + 1, 2)

            @pl.when(k + 1 < n_ch)
            def _prefetch():
                pltpu.async_copy(
                    table_hbm.at[idx_v.at[pl.ds((k + 1) * ch, ch)]],
                    rows_v.at[nxt],
                    gsem,
                )

            pltpu.make_async_copy(
                table_hbm.at[idx_v.at[pl.ds(k * ch, ch)]], rows_v.at[slot], gsem
            ).wait()
            pltpu.async_copy(
                rows_v.at[slot], out_hbm.at[pl.ds(base + k * ch, ch)], osem
            ).wait()
            return 0

        lax.fori_loop(0, n_ch, body, 0)

    return gather_kernel


def _tc_body(tok_ref, pos_ref, sid_ref, segtab_ref, gamma_ref, beta_ref, out_ref):
    x = tok_ref[...] + pos_ref[...]
    s0 = segtab_ref[0:1, :]
    s1 = segtab_ref[1:2, :]
    x = x + s0 + sid_ref[...] * (s1 - s0)
    mu = jnp.mean(x, axis=1, keepdims=True)
    xc = x - mu
    var = jnp.mean(xc * xc, axis=1, keepdims=True)
    r = lax.rsqrt(var + 1e-6)
    out_ref[...] = xc * r * gamma_ref[...] + beta_ref[...]


@functools.lru_cache(maxsize=None)
def _tc_fuse_ln(n_tokens: int, seq_len: int, d: int):
    tile = 512
    s_tiles = seq_len // tile

    return pl.pallas_call(
        _tc_body,
        grid=(n_tokens // tile,),
        in_specs=[
            pl.BlockSpec((tile, d), lambda t: (t, 0)),
            pl.BlockSpec((tile, d), lambda t: (lax.rem(t, s_tiles), 0)),
            pl.BlockSpec((tile, 1), lambda t: (t, 0)),
            pl.BlockSpec((2, d), lambda t: (0, 0)),
            pl.BlockSpec((1, d), lambda t: (0, 0)),
            pl.BlockSpec((1, d), lambda t: (0, 0)),
        ],
        out_specs=pl.BlockSpec((tile, d), lambda t: (t, 0)),
        out_shape=jax.ShapeDtypeStruct((n_tokens, d), jnp.float32),
    )


def kernel(input_ids, segment_ids, token_table, pos_table, seg_table, gamma, beta):
    b, s = input_ids.shape
    vocab, d = token_table.shape
    n = b * s

    flat_ids = input_ids.reshape(n)
    tok = _sc_gather(n, vocab, d)(flat_ids, token_table)

    sid_f = segment_ids.reshape(n, 1).astype(jnp.float32)
    out = _tc_fuse_ln(n, s, d)(
        tok,
        pos_table,
        sid_f,
        seg_table,
        gamma.reshape(1, d),
        beta.reshape(1, d),
    )
    return out.reshape(b, s, d)


# speedup vs baseline: 1.7668x; 1.0342x over previous
import functools

import jax
import jax.numpy as jnp
from jax import lax
from jax.experimental import pallas as pl
from jax.experimental.pallas import tpu as pltpu
from jax.experimental.pallas import tpu_sc as plsc


@functools.lru_cache(maxsize=None)
def _sc_gather(n_tokens: int, vocab: int, d: int):
    info = plsc.get_sparse_core_info()
    nw = info.num_cores * info.num_subcores
    per_w = n_tokens // nw
    ch = 64
    n_ch = per_w // ch
    mesh = plsc.VectorSubcoreMesh(core_axis_name="c", subcore_axis_name="s")

    @functools.partial(
        pl.kernel,
        mesh=mesh,
        out_type=jax.ShapeDtypeStruct((n_tokens, d), jnp.float32),
        scratch_types=[
            pltpu.VMEM((per_w,), jnp.int32),
            pltpu.VMEM((2, ch, d), jnp.float32),
            pltpu.SemaphoreType.DMA,
            pltpu.SemaphoreType.DMA,
        ],
    )
    def gather_kernel(ids_hbm, table_hbm, out_hbm, idx_v, rows_v, gsem, osem):
        wid = lax.axis_index("s") * info.num_cores + lax.axis_index("c")
        base = wid * per_w
        pltpu.sync_copy(ids_hbm.at[pl.ds(base, per_w)], idx_v)

        pltpu.async_copy(table_hbm.at[idx_v.at[pl.ds(0, ch)]], rows_v.at[0], gsem)

        def body(k, _):
            slot = lax.rem(k, 2)
            nxt = lax.rem(k + 1, 2)

            @pl.when(k + 1 < n_ch)
            def _prefetch():
                pltpu.async_copy(
                    table_hbm.at[idx_v.at[pl.ds((k + 1) * ch, ch)]],
                    rows_v.at[nxt],
                    gsem,
                )

            pltpu.make_async_copy(
                table_hbm.at[idx_v.at[pl.ds(k * ch, ch)]], rows_v.at[slot], gsem
            ).wait()
            pltpu.async_copy(
                rows_v.at[slot], out_hbm.at[pl.ds(base + k * ch, ch)], osem
            ).wait()
            return 0

        lax.fori_loop(0, n_ch, body, 0)

    return gather_kernel


def _tc_body(tok_ref, pos_ref, sid_ref, segtab_ref, gamma_ref, beta_ref, out_ref):
    x = tok_ref[...] + pos_ref[...]
    s0 = segtab_ref[0:1, :]
    s1 = segtab_ref[1:2, :]
    x = x + s0 + sid_ref[...] * (s1 - s0)
    mu = jnp.mean(x, axis=1, keepdims=True)
    xc = x - mu
    var = jnp.mean(xc * xc, axis=1, keepdims=True)
    r = lax.rsqrt(var + 1e-6)
    out_ref[...] = xc * r * gamma_ref[...] + beta_ref[...]


@functools.lru_cache(maxsize=None)
def _tc_fuse_ln(n_tokens: int, seq_len: int, d: int):
    tile = 512
    s_tiles = seq_len // tile
    batch = n_tokens // seq_len

    return pl.pallas_call(
        _tc_body,
        grid=(s_tiles, batch),
        in_specs=[
            pl.BlockSpec((tile, d), lambda st, b: (b * s_tiles + st, 0)),
            pl.BlockSpec((tile, d), lambda st, b: (st, 0)),
            pl.BlockSpec((tile, 1), lambda st, b: (b * s_tiles + st, 0)),
            pl.BlockSpec((2, d), lambda st, b: (0, 0)),
            pl.BlockSpec((1, d), lambda st, b: (0, 0)),
            pl.BlockSpec((1, d), lambda st, b: (0, 0)),
        ],
        out_specs=pl.BlockSpec((tile, d), lambda st, b: (b * s_tiles + st, 0)),
        out_shape=jax.ShapeDtypeStruct((n_tokens, d), jnp.float32),
    )


def kernel(input_ids, segment_ids, token_table, pos_table, seg_table, gamma, beta):
    b, s = input_ids.shape
    vocab, d = token_table.shape
    n = b * s

    flat_ids = input_ids.reshape(n)
    tok = _sc_gather(n, vocab, d)(flat_ids, token_table)

    sid_f = segment_ids.reshape(n, 1).astype(jnp.float32)
    out = _tc_fuse_ln(n, s, d)(
        tok,
        pos_table,
        sid_f,
        seg_table,
        gamma.reshape(1, d),
        beta.reshape(1, d),
    )
    return out.reshape(b, s, d)
